# Initial kernel scaffold; baseline (speedup 1.0000x reference)
import jax
import jax.numpy as jnp
from jax.experimental import pallas as pl


def kernel(points1, points2, t, W1, b1, g1, be1, W2, b2, g2, be2, W3, b3, g3, be3, k):
    raise NotImplementedError("write your pallas kernel here")


# speedup vs baseline: 4.0978x; 4.0978x over previous
import functools

import jax
import jax.numpy as jnp
from jax import lax
from jax.experimental import pallas as pl
from jax.experimental.pallas import tpu as pltpu
from jax.experimental.pallas import tpu_sc as plsc

K = 32
_I32_MAX = 0x7FFFFFFF

_NC, _NS, _L = 2, 16, 16
_NW = _NC * _NS


def _topk_body(q_ref, db_ref, idx_ref, *, rows, n):
    q = q_ref[0]
    db = db_ref[0]
    qsq = jnp.sum(q * q, axis=0)
    dbsq = jnp.sum(db * db, axis=0)
    qdb = lax.dot_general(
        q, db, (((0,), (0,)), ((), ())),
        preferred_element_type=jnp.float32)
    d2 = qsq[:, None] + dbsq[None, :] - 2.0 * qdb

    col = lax.broadcasted_iota(jnp.int32, (rows, n), 1)
    colk = lax.broadcasted_iota(jnp.int32, (rows, K), 1)
    kmin0 = jnp.min(d2, axis=1, keepdims=True)
    iacc0 = jnp.zeros((rows, K), jnp.int32)

    def step(j, carry):
        d2, kmin, iacc = carry
        sel = d2 == kmin
        iv = jnp.min(jnp.where(sel, col, n), axis=1, keepdims=True)
        iacc = jnp.where(colk == j, iv, iacc)
        d2 = jnp.where(sel, jnp.float32(jnp.inf), d2)
        kmin = jnp.min(d2, axis=1, keepdims=True)
        return d2, kmin, iacc

    _, _, iacc = lax.fori_loop(0, K, step, (d2, kmin0, iacc0))
    idx_ref[0] = iacc


def _topk(points1, dbs):
    B, _, n = points1.shape
    P = dbs.shape[0]
    rows = min(512, n)
    nb = n // rows
    return pl.pallas_call(
        functools.partial(_topk_body, rows=rows, n=n),
        grid=(P, nb),
        in_specs=[
            pl.BlockSpec((1, 3, rows), lambda p, r: (p // 2, 0, r)),
            pl.BlockSpec((1, 3, n), lambda p, r: (p, 0, 0)),
        ],
        out_specs=pl.BlockSpec((1, rows, K), lambda p, r: (p, r, 0)),
        out_shape=jax.ShapeDtypeStruct((P, n, K), jnp.int32),
        compiler_params=pltpu.CompilerParams(
            dimension_semantics=("arbitrary", "arbitrary")),
    )(points1, dbs)


def _gather(dbs, idx):
    P, _, n = dbs.shape
    k = idx.shape[2]
    cpp = max(_NW // P, 1)
    rpc = n // cpp
    sub = 2
    rs = rpc // sub

    mesh = plsc.VectorSubcoreMesh(core_axis_name="c", subcore_axis_name="s")

    @functools.partial(
        pl.kernel,
        out_type=jax.ShapeDtypeStruct((P * 3 * n * k,), jnp.float32),
        mesh=mesh,
        compiler_params=pltpu.CompilerParams(
            needs_layout_passes=False, use_tc_tiling_on_sc=False),
        scratch_types=[
            pltpu.VMEM((n,), jnp.float32),
            pltpu.VMEM((n,), jnp.float32),
            pltpu.VMEM((n,), jnp.float32),
            pltpu.VMEM((rs * k,), jnp.int32),
            pltpu.VMEM((rs * k,), jnp.float32),
            pltpu.VMEM((rs * k,), jnp.float32),
            pltpu.VMEM((rs * k,), jnp.float32),
        ],
    )
    def body(dbs_hbm, idx_hbm, out_hbm, tx_v, ty_v, tz_v,
             idx_v, vx_v, vy_v, vz_v):
        w = lax.axis_index("s") * _NC + lax.axis_index("c")
        p = w // cpp
        q = w % cpp
        tabs = (tx_v, ty_v, tz_v)
        vals = (vx_v, vy_v, vz_v)
        for c in range(3):
            pltpu.sync_copy(dbs_hbm.at[pl.ds((p * 3 + c) * n, n)], tabs[c])

        def do_sub(si, _):
            r0 = q * rpc + si * rs
            pltpu.sync_copy(idx_hbm.at[pl.ds((p * n + r0) * k, rs * k)], idx_v)

            def grp(i, _):
                vi = idx_v[pl.ds(i * _L, _L)]
                for c in range(3):
                    vals[c][pl.ds(i * _L, _L)] = plsc.load_gather(tabs[c], [vi])
                return 0

            lax.fori_loop(0, rs * k // _L, grp, 0)
            for c in range(3):
                pltpu.sync_copy(
                    vals[c], out_hbm.at[pl.ds(((p * 3 + c) * n + r0) * k, rs * k)])
            return 0

        lax.fori_loop(0, sub, do_sub, 0)

    out = body(dbs.reshape(-1), idx.reshape(-1))
    return out.reshape(P, 3, n, k)


def _mlp_body(p1_ref, gpp_ref, W1_ref, B1_ref, G1_ref, E1_ref,
              W2_ref, B2_ref, G2_ref, E2_ref,
              W3_ref, B3_ref, G3_ref, E3_ref,
              out_ref, stats_ref, *, nblk, m_total):
    ph = pl.program_id(0)
    b = pl.program_id(1)
    nbi = pl.program_id(2)

    @pl.when(jnp.logical_and(ph == 0,
                             jnp.logical_and(b == 0, nbi == 0)))
    def _init():
        stats_ref[...] = jnp.zeros_like(stats_ref)

    q = p1_ref[0]
    nn = jnp.concatenate([gpp_ref[0], gpp_ref[1]], axis=2)
    resi = nn - q[:, :, None]
    rx, ry, rz = resi[0], resi[1], resi[2]
    dist = jnp.sqrt(rx * rx + ry * ry + rz * rz)
    M = nblk * 2 * K
    x0 = jnp.stack([rx, ry, rz, dist], axis=-1).reshape(M, 4)

    eps = jnp.float32(1e-3)
    mt = jnp.float32(m_total)

    def dot(x, w_ref):
        return lax.dot_general(
            x, w_ref[...], (((1,), (1,)), ((), ())),
            preferred_element_type=jnp.float32)

    def norm(z, srow, qrow, c, g_ref, e_ref):
        s = stats_ref[srow:srow + 1, 0:c]
        sq = stats_ref[qrow:qrow + 1, 0:c]
        mu = s / mt
        var = sq / mt - mu * mu
        inv = 1.0 / jnp.sqrt(var + eps)
        return jnp.maximum(
            g_ref[...][:, 0:c] * (z - mu) * inv + e_ref[...][:, 0:c], 0.0)

    z1 = dot(x0, W1_ref) + B1_ref[...]

    @pl.when(ph == 0)
    def _acc1():
        stats_ref[0:1, 0:64] = stats_ref[0:1, 0:64] + jnp.sum(z1, axis=0)[None, :]
        stats_ref[1:2, 0:64] = stats_ref[1:2, 0:64] + jnp.sum(z1 * z1, axis=0)[None, :]

    @pl.when(ph >= 1)
    def _rest():
        a1 = norm(z1, 0, 1, 64, G1_ref, E1_ref)
        z2 = dot(a1, W2_ref) + B2_ref[...]

        @pl.when(ph == 1)
        def _acc2():
            stats_ref[2:3, 0:64] = stats_ref[2:3, 0:64] + jnp.sum(z2, axis=0)[None, :]
            stats_ref[3:4, 0:64] = stats_ref[3:4, 0:64] + jnp.sum(z2 * z2, axis=0)[None, :]

        @pl.when(ph >= 2)
        def _l3():
            a2 = norm(z2, 2, 3, 64, G2_ref, E2_ref)
            z3 = dot(a2, W3_ref) + B3_ref[...]

            @pl.when(ph == 2)
            def _acc3():
                stats_ref[4:5, :] = stats_ref[4:5, :] + jnp.sum(z3, axis=0)[None, :]
                stats_ref[5:6, :] = stats_ref[5:6, :] + jnp.sum(z3 * z3, axis=0)[None, :]

            @pl.when(ph == 3)
            def _fin():
                a3 = norm(z3, 4, 5, 128, G3_ref, E3_ref)
                m = jnp.max(a3, axis=1).reshape(nblk, 2 * K)
                mm = jnp.max(m, axis=1, keepdims=True)
                e = jnp.exp(m - mm)
                w = e / jnp.sum(e, axis=1, keepdims=True)
                for c in range(3):
                    out_ref[0, 0, c, :] = jnp.sum(w * nn[c], axis=1)


def _mlp(points1, gpp, W1, b1, g1, be1, W2, b2, g2, be2, W3, b3, g3, be3):
    B, _, n = points1.shape
    nblk = min(128, n)
    nb = n // nblk
    m_total = B * n * 2 * K
    row = lambda v: v.reshape(1, -1)
    return pl.pallas_call(
        functools.partial(_mlp_body, nblk=nblk, m_total=m_total),
        grid=(4, B, nb),
        in_specs=[
            pl.BlockSpec((1, 3, nblk), lambda p, b, r: (b, 0, r)),
            pl.BlockSpec((2, 3, nblk, K), lambda p, b, r: (b, 0, r, 0)),
        ] + [
            pl.BlockSpec(s, lambda p, b, r: (0, 0))
            for s in ((64, 4), (1, 64), (1, 64), (1, 64),
                      (64, 64), (1, 64), (1, 64), (1, 64),
                      (128, 64), (1, 128), (1, 128), (1, 128))
        ],
        out_specs=pl.BlockSpec((1, 1, 3, nblk), lambda p, b, r: (p, b, 0, r)),
        out_shape=jax.ShapeDtypeStruct((4, B, 3, n), jnp.float32),
        scratch_shapes=[pltpu.VMEM((8, 128), jnp.float32)],
        compiler_params=pltpu.CompilerParams(
            dimension_semantics=("arbitrary", "arbitrary", "arbitrary")),
    )(points1, gpp,
      W1, row(b1), row(g1), row(be1),
      W2, row(b2), row(g2), row(be2),
      W3, row(b3), row(g3), row(be3))[3]


def kernel(points1, points2, t, W1, b1, g1, be1, W2, b2, g2, be2,
           W3, b3, g3, be3, k):
    B, _, n = points1.shape
    dbs = jnp.stack([points1, points2], axis=1).reshape(2 * B, 3, n)
    idx = _topk(points1, dbs)
    gpp = _gather(dbs, idx)
    return _mlp(points1, gpp, W1, b1, g1, be1, W2, b2, g2, be2,
                W3, b3, g3, be3)


# speedup vs baseline: 5.7675x; 1.4075x over previous
import functools

import jax
import jax.numpy as jnp
from jax import lax
from jax.experimental import pallas as pl
from jax.experimental.pallas import tpu as pltpu
from jax.experimental.pallas import tpu_sc as plsc

K = 32

_NC, _NS, _L = 2, 16, 16
_NW = _NC * _NS


def _topk_body(q_ref, db_ref, idx_ref, *, rows, n):
    q = q_ref[0]
    db = db_ref[0]
    qsq = jnp.sum(q * q, axis=0)
    dbsq = jnp.sum(db * db, axis=0)
    qdb = lax.dot_general(
        q, db, (((0,), (0,)), ((), ())),
        preferred_element_type=jnp.float32)
    d2 = qsq[:, None] + dbsq[None, :] - 2.0 * qdb

    col = lax.broadcasted_iota(jnp.int32, (rows, n), 1)
    colk = lax.broadcasted_iota(jnp.int32, (rows, K), 1)
    kmin0 = jnp.min(d2, axis=1, keepdims=True)
    iacc0 = jnp.zeros((rows, K), jnp.int32)

    def step(j, carry):
        d2, kmin, iacc = carry
        sel = d2 == kmin
        iv = jnp.min(jnp.where(sel, col, n), axis=1, keepdims=True)
        iacc = jnp.where(colk == j, iv, iacc)
        d2 = jnp.where(sel, jnp.float32(jnp.inf), d2)
        kmin = jnp.min(d2, axis=1, keepdims=True)
        return d2, kmin, iacc

    _, _, iacc = lax.fori_loop(0, K, step, (d2, kmin0, iacc0))
    idx_ref[0] = iacc


def _topk(points1, dbs):
    B, _, n = points1.shape
    P = dbs.shape[0]
    rows = min(512, n)
    nb = n // rows
    return pl.pallas_call(
        functools.partial(_topk_body, rows=rows, n=n),
        grid=(P, nb),
        in_specs=[
            pl.BlockSpec((1, 3, rows), lambda p, r: (p // 2, 0, r)),
            pl.BlockSpec((1, 3, n), lambda p, r: (p, 0, 0)),
        ],
        out_specs=pl.BlockSpec((1, rows, K), lambda p, r: (p, r, 0)),
        out_shape=jax.ShapeDtypeStruct((P, n, K), jnp.int32),
        compiler_params=pltpu.CompilerParams(
            dimension_semantics=("arbitrary", "arbitrary")),
    )(points1, dbs)


def _gather(p1f, dbsf, idx, B, n):
    P = 2 * B
    k = idx.shape[2]
    E = P * n * k
    cpp = max(_NW // P, 1)
    rpc = n // cpp
    sub = max(rpc // 256, 1)
    rs = rpc // sub

    mesh = plsc.VectorSubcoreMesh(core_axis_name="c", subcore_axis_name="s")
    plane = jax.ShapeDtypeStruct((E,), jnp.float32)

    @functools.partial(
        pl.kernel,
        out_type=(plane,) * 7,
        mesh=mesh,
        compiler_params=pltpu.CompilerParams(
            needs_layout_passes=False, use_tc_tiling_on_sc=False),
        scratch_types=[pltpu.VMEM((n,), jnp.float32)] * 6
                      + [pltpu.VMEM((rs * k,), jnp.int32)]
                      + [pltpu.VMEM((rs * k,), jnp.float32)] * 7,
    )
    def body(p1_hbm, dbs_hbm, idx_hbm,
             rx_hbm, ry_hbm, rz_hbm, ds_hbm, nx_hbm, ny_hbm, nz_hbm,
             tx_v, ty_v, tz_v, qx_v, qy_v, qz_v, idx_v,
             rx_v, ry_v, rz_v, ds_v, nx_v, ny_v, nz_v):
        w = lax.axis_index("s") * _NC + lax.axis_index("c")
        p = w // cpp
        cq = w % cpp
        b = p // 2
        s = p % 2
        tabs = (tx_v, ty_v, tz_v)
        qtabs = (qx_v, qy_v, qz_v)
        for c in range(3):
            pltpu.sync_copy(dbs_hbm.at[pl.ds((p * 3 + c) * n, n)], tabs[c])
            pltpu.sync_copy(p1_hbm.at[pl.ds((b * 3 + c) * n, n)], qtabs[c])

        def do_sub(si, _):
            r0 = cq * rpc + si * rs
            pltpu.sync_copy(idx_hbm.at[pl.ds((p * n + r0) * k, rs * k)], idx_v)

            def row(i, _):
                nrow = jnp.full((_L,), r0 + i, jnp.int32)
                qx = plsc.load_gather(qx_v, [nrow])
                qy = plsc.load_gather(qy_v, [nrow])
                qz = plsc.load_gather(qz_v, [nrow])
                for g in range(k // _L):
                    o = i * k + g * _L
                    vi = idx_v[pl.ds(o, _L)]
                    nx = plsc.load_gather(tx_v, [vi])
                    ny = plsc.load_gather(ty_v, [vi])
                    nz = plsc.load_gather(tz_v, [vi])
                    rx = nx - qx
                    ry = ny - qy
                    rz = nz - qz
                    nx_v[pl.ds(o, _L)] = nx
                    ny_v[pl.ds(o, _L)] = ny
                    nz_v[pl.ds(o, _L)] = nz
                    rx_v[pl.ds(o, _L)] = rx
                    ry_v[pl.ds(o, _L)] = ry
                    rz_v[pl.ds(o, _L)] = rz
                    ds_v[pl.ds(o, _L)] = rx * rx + ry * ry + rz * rz
                return 0

            lax.fori_loop(0, rs, row, 0)
            base = (((s * B + b) * n) + r0) * k
            outs = (rx_hbm, ry_hbm, rz_hbm, ds_hbm, nx_hbm, ny_hbm, nz_hbm)
            bufs = (rx_v, ry_v, rz_v, ds_v, nx_v, ny_v, nz_v)
            for o_h, o_v in zip(outs, bufs):
                pltpu.sync_copy(o_v, o_h.at[pl.ds(base, rs * k)])
            return 0

        lax.fori_loop(0, sub, do_sub, 0)

    return body(p1f, dbsf, idx.reshape(-1))


def _feat_block(rx_ref, ry_ref, rz_ref, ds_ref):
    return jnp.concatenate(
        [rx_ref[...], ry_ref[...], rz_ref[...], jnp.sqrt(ds_ref[...])],
        axis=0)


def _scale_shift(stats_ref, scol, qcol, c, g_ref, e_ref, mt):
    s = stats_ref[0:c, scol:scol + 1]
    q = stats_ref[0:c, qcol:qcol + 1]
    mu = s / mt
    var = q / mt - mu * mu
    inv = 1.0 / jnp.sqrt(var + 1e-3)
    sc = g_ref[...] * inv
    sh = e_ref[...] - mu * sc
    return sc, sh


def _cdot(w_ref, x):
    return lax.dot_general(w_ref[...], x, (((1,), (0,)), ((), ())),
                           preferred_element_type=jnp.float32)


def _stats_body(rx_ref, ry_ref, rz_ref, ds_ref,
                W1_ref, B1_ref, G1_ref, E1_ref,
                W2_ref, B2_ref, G2_ref, E2_ref,
                W3_ref, B3_ref,
                out_ref, stats_ref, *, m_total):
    ph = pl.program_id(0)
    blk = pl.program_id(1)

    @pl.when(jnp.logical_and(ph == 0, blk == 0))
    def _init():
        stats_ref[...] = jnp.zeros_like(stats_ref)

    x = _feat_block(rx_ref, ry_ref, rz_ref, ds_ref)
    mt = jnp.float32(m_total)
    z1 = _cdot(W1_ref, x) + B1_ref[...]

    @pl.when(ph == 0)
    def _acc1():
        stats_ref[0:64, 0:1] += jnp.sum(z1, axis=1, keepdims=True)
        stats_ref[0:64, 1:2] += jnp.sum(z1 * z1, axis=1, keepdims=True)

    @pl.when(ph >= 1)
    def _rest():
        sc1, sh1 = _scale_shift(stats_ref, 0, 1, 64, G1_ref, E1_ref, mt)
        a1 = jnp.maximum(z1 * sc1 + sh1, 0.0)
        z2 = _cdot(W2_ref, a1) + B2_ref[...]

        @pl.when(ph == 1)
        def _acc2():
            stats_ref[0:64, 2:3] += jnp.sum(z2, axis=1, keepdims=True)
            stats_ref[0:64, 3:4] += jnp.sum(z2 * z2, axis=1, keepdims=True)

        @pl.when(ph == 2)
        def _acc3():
            sc2, sh2 = _scale_shift(stats_ref, 2, 3, 64, G2_ref, E2_ref, mt)
            a2 = jnp.maximum(z2 * sc2 + sh2, 0.0)
            z3 = _cdot(W3_ref, a2) + B3_ref[...]
            stats_ref[0:128, 4:5] += jnp.sum(z3, axis=1, keepdims=True)
            stats_ref[0:128, 5:6] += jnp.sum(z3 * z3, axis=1, keepdims=True)

    out_ref[...] = stats_ref[...]


def _stats(planes, W1, b1, g1, be1, W2, b2, g2, be2, W3, b3, m_total):
    E = planes[0].shape[0]
    EB = 16384
    nbl = E // EB
    flat = lambda v: v.reshape(1, E)
    col = lambda v: v.reshape(-1, 1)
    wspecs = [pl.BlockSpec(s, lambda p, j: (0, 0))
              for s in ((64, 4), (64, 1), (64, 1), (64, 1),
                        (64, 64), (64, 1), (64, 1), (64, 1),
                        (128, 64), (128, 1))]
    return pl.pallas_call(
        functools.partial(_stats_body, m_total=m_total),
        grid=(3, nbl),
        in_specs=[pl.BlockSpec((1, EB), lambda p, j: (0, j))] * 4 + wspecs,
        out_specs=pl.BlockSpec((128, 8), lambda p, j: (0, 0)),
        out_shape=jax.ShapeDtypeStruct((128, 8), jnp.float32),
        scratch_shapes=[pltpu.VMEM((128, 8), jnp.float32)],
        compiler_params=pltpu.CompilerParams(
            dimension_semantics=("arbitrary", "arbitrary")),
    )(flat(planes[0]), flat(planes[1]), flat(planes[2]), flat(planes[3]),
      W1, col(b1), col(g1), col(be1),
      W2, col(b2), col(g2), col(be2),
      W3, col(b3))


def _fuse_body(rxs_ref, rys_ref, rzs_ref, dss_ref,
               rxc_ref, ryc_ref, rzc_ref, dsc_ref,
               nxs_ref, nys_ref, nzs_ref, nxc_ref, nyc_ref, nzc_ref,
               st_ref,
               W1_ref, B1_ref, G1_ref, E1_ref,
               W2_ref, B2_ref, G2_ref, E2_ref,
               W3_ref, B3_ref, G3_ref, E3_ref,
               out_ref, *, nblk, m_total):
    mt = jnp.float32(m_total)
    xs = _feat_block(rxs_ref, rys_ref, rzs_ref, dss_ref)
    xc = _feat_block(rxc_ref, ryc_ref, rzc_ref, dsc_ref)
    x = jnp.concatenate([xs, xc], axis=1)

    sc1, sh1 = _scale_shift(st_ref, 0, 1, 64, G1_ref, E1_ref, mt)
    sc2, sh2 = _scale_shift(st_ref, 2, 3, 64, G2_ref, E2_ref, mt)
    sc3, sh3 = _scale_shift(st_ref, 4, 5, 128, G3_ref, E3_ref, mt)

    a1 = jnp.maximum(_cdot(W1_ref, x) * sc1 + (B1_ref[...] * sc1 + sh1), 0.0)
    a2 = jnp.maximum(_cdot(W2_ref, a1) * sc2 + (B2_ref[...] * sc2 + sh2), 0.0)
    a3 = jnp.maximum(_cdot(W3_ref, a2) * sc3 + (B3_ref[...] * sc3 + sh3), 0.0)

    ek = nblk * K
    m = jnp.max(a3, axis=0, keepdims=True)
    erow = lax.broadcasted_iota(jnp.int32, (2 * ek, nblk), 0)
    jcol = lax.broadcasted_iota(jnp.int32, (2 * ek, nblk), 1)
    ssum = ((erow % ek) // K == jcol).astype(jnp.float32)
    e = jnp.exp(m)
    segsum = lax.dot_general(e, ssum, (((1,), (0,)), ((), ())),
                             preferred_element_type=jnp.float32)
    den = lax.dot_general(segsum, ssum, (((1,), (1,)), ((), ())),
                          preferred_element_type=jnp.float32)
    w = e / den
    for c, (nns_ref, nnc_ref) in enumerate(
            ((nxs_ref, nxc_ref), (nys_ref, nyc_ref), (nzs_ref, nzc_ref))):
        nnf = jnp.concatenate([nns_ref[...], nnc_ref[...]], axis=1)
        fused = lax.dot_general(w * nnf, ssum, (((1,), (0,)), ((), ())),
                                preferred_element_type=jnp.float32)
        out_ref[0, 0, c, :] = fused[0]


def _fuse(planes, stats, W1, b1, g1, be1, W2, b2, g2, be2, W3, b3, g3, be3,
          B, n, m_total):
    nblk = min(128, n)
    nbn = n // nblk
    E = planes[0].shape[0]
    BN = B * n
    ek = nblk * K
    flat = lambda v: v.reshape(1, E)
    col = lambda v: v.reshape(-1, 1)

    fs = pl.BlockSpec((1, ek), lambda b, j: (0, b * nbn + j))
    fc = pl.BlockSpec((1, ek), lambda b, j: (0, BN // nblk + b * nbn + j))
    wspecs = [pl.BlockSpec(s, lambda b, j: (0, 0))
              for s in ((128, 8),
                        (64, 4), (64, 1), (64, 1), (64, 1),
                        (64, 64), (64, 1), (64, 1), (64, 1),
                        (128, 64), (128, 1), (128, 1), (128, 1))]
    out = pl.pallas_call(
        functools.partial(_fuse_body, nblk=nblk, m_total=m_total),
        grid=(B, nbn),
        in_specs=[fs, fs, fs, fs, fc, fc, fc, fc, fs, fs, fs, fc, fc, fc]
                 + wspecs,
        out_specs=pl.BlockSpec((1, 1, 3, nblk), lambda b, j: (b, j, 0, 0)),
        out_shape=jax.ShapeDtypeStruct((B, nbn, 3, nblk), jnp.float32),
        compiler_params=pltpu.CompilerParams(
            dimension_semantics=("arbitrary", "arbitrary")),
    )(flat(planes[0]), flat(planes[1]), flat(planes[2]), flat(planes[3]),
      flat(planes[0]), flat(planes[1]), flat(planes[2]), flat(planes[3]),
      flat(planes[4]), flat(planes[5]), flat(planes[6]),
      flat(planes[4]), flat(planes[5]), flat(planes[6]),
      stats,
      W1, col(b1), col(g1), col(be1),
      W2, col(b2), col(g2), col(be2),
      W3, col(b3), col(g3), col(be3))
    return out.transpose(0, 2, 1, 3).reshape(B, 3, n)


def kernel(points1, points2, t, W1, b1, g1, be1, W2, b2, g2, be2,
           W3, b3, g3, be3, k):
    B, _, n = points1.shape
    dbs = jnp.stack([points1, points2], axis=1).reshape(2 * B, 3, n)
    idx = _topk(points1, dbs)
    planes = _gather(points1.reshape(-1), dbs.reshape(-1), idx, B, n)
    m_total = B * n * 2 * K
    stats = _stats(planes, W1, b1, g1, be1, W2, b2, g2, be2, W3, b3, m_total)
    return _fuse(planes, stats, W1, b1, g1, be1, W2, b2, g2, be2,
                 W3, b3, g3, be3, B, n, m_total)
